# Initial kernel scaffold; baseline (speedup 1.0000x reference)
import jax
import jax.numpy as jnp
from jax.experimental import pallas as pl


def kernel(x, node_deg, pos, edge_vec, edge_len, edge_index, params):
    raise NotImplementedError("write your pallas kernel here")


# speedup vs baseline: 2.2833x; 2.2833x over previous
import functools
import math

import jax
import jax.numpy as jnp
import numpy as np
from jax.experimental import pallas as pl
from jax.experimental.pallas import tpu as pltpu

SCC = 32
VCC = 32
FD = SCC + 3 * VCC
NLAYERS = 2
EB = 2560

_SQ3 = math.sqrt(3.0)
_NORM = 1.0 / 8.0


def _edge_tp_kernel(gath_ref, e1_ref, elen_ref, wss_ref, wvv_ref, wsv_ref,
                    wvs_ref, wr_ref, br_ref, out_ref):
    g = gath_ref[...]
    ss = g[:, :SCC]
    sv0 = g[:, SCC:SCC + VCC]
    sv1 = g[:, SCC + VCC:SCC + 2 * VCC]
    sv2 = g[:, SCC + 2 * VCC:]
    e1 = e1_ref[...]
    e1x = e1[:, 0:1]
    e1y = e1[:, 1:2]
    e1z = e1[:, 2:3]
    dot = (sv0 * e1x + sv1 * e1y + sv2 * e1z) * (1.0 / _SQ3)
    ms = (ss @ wss_ref[...] + dot @ wvv_ref[...]) * _NORM
    p = ss @ wsv_ref[...]
    mv0 = (p * e1x + sv0 @ wvs_ref[...]) * _NORM
    mv1 = (p * e1y + sv1 @ wvs_ref[...]) * _NORM
    mv2 = (p * e1z + sv2 @ wvs_ref[...]) * _NORM
    scale = jax.nn.sigmoid(elen_ref[...] @ wr_ref[...] + br_ref[...])
    out_ref[...] = jnp.concatenate([ms, mv0, mv1, mv2], axis=1) * scale


def _edge_messages(gathered, e1pad, edge_len, wss, wvv, wsv, wvs, wr, br):
    e = gathered.shape[0]
    grid = (e // EB,)
    return pl.pallas_call(
        _edge_tp_kernel,
        grid=grid,
        in_specs=[
            pl.BlockSpec((EB, FD), lambda i: (i, 0)),
            pl.BlockSpec((EB, 8), lambda i: (i, 0)),
            pl.BlockSpec((EB, 50), lambda i: (i, 0)),
            pl.BlockSpec((SCC, SCC), lambda i: (0, 0)),
            pl.BlockSpec((SCC, SCC), lambda i: (0, 0)),
            pl.BlockSpec((SCC, SCC), lambda i: (0, 0)),
            pl.BlockSpec((SCC, SCC), lambda i: (0, 0)),
            pl.BlockSpec((50, FD), lambda i: (0, 0)),
            pl.BlockSpec((1, FD), lambda i: (0, 0)),
        ],
        out_specs=pl.BlockSpec((EB, FD), lambda i: (i, 0)),
        out_shape=jax.ShapeDtypeStruct((e, FD), jnp.float32),
    )(gathered, e1pad, edge_len, wss, wvv, wsv, wvs, wr, br)


def _perm_cols():
    perm = list(range(SCC))
    for i in range(3):
        for u in range(VCC):
            perm.append(SCC + u * 3 + i)
    return np.array(perm, np.int32)


def _process(s, vcm, gamma, beta, wg, bg):
    mu = s.mean(-1, keepdims=True)
    var = s.var(-1, keepdims=True)
    s = (s - mu) / jnp.sqrt(var + 1e-5) * gamma + beta
    s = jax.nn.silu(s)
    gate = jax.nn.sigmoid(s @ wg + bg)
    vcm = vcm * jnp.tile(gate, (1, 3))
    return s, vcm


def kernel(x, node_deg, pos, edge_vec, edge_len, edge_index, params):
    n = x.shape[0]
    p = params
    perm = _perm_cols()

    scalar_attr = jnp.concatenate([x, node_deg], axis=-1)
    s = scalar_attr @ (p['W_embed_s'] / jnp.sqrt(float(p['W_embed_s'].shape[0])))
    vcm = (pos[:, :, None] * p['W_embed_v'][0][None, None, :]).reshape(n, 3 * VCC)
    s, vcm = _process(s, vcm, p['ln_g_0'], p['ln_b_0'], p['Wg_0'], p['bg_0'])

    unit = edge_vec / (jnp.linalg.norm(edge_vec, axis=-1, keepdims=True) + 1e-12)
    e1 = _SQ3 * unit[:, jnp.array([1, 2, 0])]
    e1pad = jnp.pad(e1, ((0, 0), (0, 5)))
    src, dst = edge_index[0], edge_index[1]

    for i in range(NLAYERS):
        feat = jnp.concatenate([s, vcm], axis=-1)
        gathered = feat[src]
        wr = p[f'Wr_{i}'][:, perm]
        br = p[f'br_{i}'][perm][None, :]
        msg = _edge_messages(gathered, e1pad, edge_len,
                             p[f'w_ss_{i}'], p[f'w_vv_{i}'], p[f'w_sv_{i}'],
                             p[f'w_vs_{i}'], wr, br)
        agg = jax.ops.segment_sum(msg, dst, num_segments=n)
        hs = s + agg[:, :SCC]
        hvcm = vcm + agg[:, SCC:]
        wl_s = p[f'Wl_s_{i}'] / jnp.sqrt(float(SCC))
        wl_v = p[f'Wl_v_{i}'] / jnp.sqrt(float(VCC))
        s = hs @ wl_s
        vcm = (hvcm.reshape(n, 3, VCC) @ wl_v).reshape(n, 3 * VCC)
        s, vcm = _process(s, vcm, p[f'ln_g_{i+1}'], p[f'ln_b_{i+1}'],
                          p[f'Wg_{i+1}'], p[f'bg_{i+1}'])

    s = s @ (p['Wo_s'] / jnp.sqrt(float(SCC)))
    vcm = (vcm.reshape(n, 3, VCC) @ (p['Wo_v'] / jnp.sqrt(float(VCC)))).reshape(n, 3 * VCC)
    s, vcm = _process(s, vcm, p['ln_g_3'], p['ln_b_3'], p['Wg_3'], p['bg_3'])

    scalar_context = s @ p['Wst'] + p['bst']
    toks = []
    for axis in range(3):
        comp = vcm[:, axis * VCC:(axis + 1) * VCC]
        toks.append(comp @ p['Wvt'] + p['bvt'] + scalar_context)
    tokens = jnp.stack(toks, axis=1).reshape(-1, 64)
    return tokens
